# Initial kernel scaffold; baseline (speedup 1.0000x reference)
import jax
import jax.numpy as jnp
from jax.experimental import pallas as pl


def kernel(input_emb, position, pe):
    raise NotImplementedError("write your pallas kernel here")


# speedup vs baseline: 1.2653x; 1.2653x over previous
import functools

import jax
import jax.numpy as jnp
from jax import lax
from jax.experimental import pallas as pl
from jax.experimental.pallas import tpu as pltpu
from jax.experimental.pallas import tpu_sc as plsc

MAX_LEN = 1000
HIDDEN_DIM = 128

NW = 32
ROWS = 16 * 64 * 50
ROWS_PER_W = ROWS // NW
CHUNK = 80
NCHUNK = ROWS_PER_W // CHUNK
LANES = 16
VECS_PER_ROW = HIDDEN_DIM // LANES


def _sc_kernel(emb_hbm, pos_hbm, pe_hbm, out_hbm,
               idx_v, in_v, pe_v, sem_in, sem_pe, sem_out):
  wid = lax.axis_index("s") * 2 + lax.axis_index("c")
  pltpu.sync_copy(pos_hbm.at[wid], idx_v)

  def chunk_body(c, _):
    base = wid * ROWS_PER_W + c * CHUNK
    in_cp = pltpu.async_copy(emb_hbm.at[pl.ds(base, CHUNK)], in_v, sem_in)
    pe_cp = pltpu.async_copy(pe_hbm.at[idx_v.at[c]], pe_v, sem_pe)
    in_cp.wait()
    pe_cp.wait()

    def row_body(r, _):
      for j in range(VECS_PER_ROW):
        sl = pl.ds(j * LANES, LANES)
        plsc.addupdate(in_v.at[r, sl], pe_v[r, sl])
      return 0

    lax.fori_loop(0, CHUNK, row_body, 0)
    pltpu.async_copy(in_v, out_hbm.at[pl.ds(base, CHUNK)], sem_out).wait()
    return 0

  lax.fori_loop(0, NCHUNK, chunk_body, 0)


def kernel(input_emb, position, pe):
  B, N, L, D = input_emb.shape
  emb2d = input_emb.reshape(ROWS, D)
  pos2d = position.reshape(NW, NCHUNK, CHUNK).astype(jnp.int32)

  run = functools.partial(
      pl.kernel,
      mesh=plsc.VectorSubcoreMesh(core_axis_name="c", subcore_axis_name="s"),
      out_type=jax.ShapeDtypeStruct((ROWS, D), jnp.float32),
      scratch_types=[
          pltpu.VMEM((NCHUNK, CHUNK), jnp.int32),
          pltpu.VMEM((CHUNK, D), jnp.float32),
          pltpu.VMEM((CHUNK, D), jnp.float32),
          pltpu.SemaphoreType.DMA,
          pltpu.SemaphoreType.DMA,
          pltpu.SemaphoreType.DMA,
      ],
  )(_sc_kernel)

  out = run(emb2d, pos2d, pe)
  return out.reshape(B, N, L, D)
